# baseline (speedup 1.0000x reference)
import jax
import jax.numpy as jnp
from jax import lax
from jax.experimental import pallas as pl
from jax.experimental.pallas import tpu as pltpu
from jax.experimental.pallas import tpu_sc as plsc

_N = 10000
_NPAD = 10240
_D = 128
_R = 200
_RPAD = 256
_NSUB = 16
_CHUNK = 80
_STRIPE = _NPAD // _NSUB


def _fill(buf, val):
    def body(i, _):
        r = i >> 3
        kk = (i & 7) * 16
        buf[r, pl.ds(kk, 16)] = val
        return 0
    lax.fori_loop(0, _CHUNK * 8, body, 0)


def _sc_acc_body(ei, ea, x_hbm, rel_hbm, acc_hbm,
                 a_sp, rel_sp,
                 ridx0, cidx0, tidx0, xbuf0, rbuf0,
                 ridx1, cidx1, tidx1, xbuf1, rbuf1,
                 gx0, gr0, sc0, gx1, gr1, sc1):
    c = lax.axis_index("c")
    s = lax.axis_index("s")
    num_e = ei.shape[0] // 2
    half = num_e // 2
    per_sub = half // _NSUB
    n_chunks = per_sub // _CHUNK
    last = n_chunks - 1
    zero16 = jnp.zeros((16,), jnp.float32)
    sets = ((ridx0, cidx0, tidx0, xbuf0, rbuf0, gx0, gr0, sc0),
            (ridx1, cidx1, tidx1, xbuf1, rbuf1, gx1, gr1, sc1))

    def _mul(xb, rb):
        def body(r4, _):
            r = r4 * 4
            for dr in range(4):
                for kk in range(8):
                    sl = pl.ds(kk * 16, 16)
                    xb[r + dr, sl] = xb[r + dr, sl] * rb[r + dr, sl]
            return 0
        lax.fori_loop(0, _CHUNK // 4, body, 0)

    def _load_idx(j, ci):
        ridx, cidx, tidx = sets[j][0], sets[j][1], sets[j][2]
        off = ebase + ci * _CHUNK
        pltpu.sync_copy(ei.at[pl.ds(off, _CHUNK)], ridx)
        pltpu.sync_copy(ei.at[pl.ds(num_e + off, _CHUNK)], cidx)
        pltpu.sync_copy(ea.at[pl.ds(off, _CHUNK)], tidx)

    def _issue_gathers(j):
        _, cidx, tidx, xb, rb, gx, gr, _sc = sets[j]
        pltpu.async_copy(x_hbm.at[cidx], xb, gx)
        pltpu.async_copy(rel_sp.at[tidx], rb, gr)

    def _wait_gathers(j):
        _, cidx, tidx, xb, rb, gx, gr, _sc = sets[j]
        pltpu.make_async_copy(x_hbm.at[cidx], xb, gx).wait()
        pltpu.make_async_copy(rel_sp.at[tidx], rb, gr).wait()

    def _issue_scatter(j):
        ridx, _, _, xb, _, _, _, sc = sets[j]
        pltpu.async_copy(xb, a_sp.at[ridx], sc, add=True)

    def _wait_scatter(j):
        ridx, _, _, xb, _, _, _, sc = sets[j]
        pltpu.make_async_copy(xb, a_sp.at[ridx], sc).wait()

    pltpu.sync_copy(rel_hbm.at[pl.ds(s * (_RPAD // _NSUB), _RPAD // _NSUB)],
                    rel_sp.at[pl.ds(s * (_RPAD // _NSUB), _RPAD // _NSUB)])
    _fill(xbuf0, zero16)
    r0 = s * _STRIPE
    for b in range(_STRIPE // _CHUNK):
        pltpu.sync_copy(xbuf0, a_sp.at[pl.ds(r0 + b * _CHUNK, _CHUNK)])
    plsc.subcore_barrier()

    ebase = c * half + s * per_sub

    _load_idx(0, 0)
    _issue_gathers(0)
    _load_idx(1, 1)
    _issue_gathers(1)

    def _pair(i, _):
        for j in (0, 1):
            _wait_gathers(j)
            _mul(sets[j][3], sets[j][4])
            _issue_scatter(j)
        for j in (0, 1):
            p = jnp.minimum(2 * i + 2 + j, last)
            _wait_scatter(j)
            _load_idx(j, p)
            _issue_gathers(j)
        return 0
    lax.fori_loop(0, (n_chunks - 1) // 2, _pair, 0)

    _wait_gathers(0)
    _mul(xbuf0, rbuf0)
    _issue_scatter(0)
    _wait_scatter(0)
    _wait_gathers(1)

    plsc.subcore_barrier()
    pltpu.sync_copy(a_sp.at[pl.ds(r0, _STRIPE)],
                    acc_hbm.at[c, pl.ds(r0, _STRIPE)])


def _sc_deg_body(rows, deg_hbm,
                 d_sp, ridx0, ridx1, onesbuf, s0, s1):
    c = lax.axis_index("c")
    s = lax.axis_index("s")
    half = rows.shape[0] // 2
    per_sub = half // _NSUB
    n_chunks = per_sub // _CHUNK
    last = n_chunks - 1
    zero16 = jnp.zeros((16,), jnp.float32)
    one16 = jnp.ones((16,), jnp.float32)
    sems = (s0, s1)
    idxs = (ridx0, ridx1)

    _fill(onesbuf, zero16)
    r0 = s * _STRIPE
    for b in range(_STRIPE // _CHUNK):
        pltpu.sync_copy(onesbuf, d_sp.at[pl.ds(r0 + b * _CHUNK, _CHUNK)])
    _fill(onesbuf, one16)
    plsc.subcore_barrier()

    ebase = c * half + s * per_sub

    def _load_ridx(j, ci):
        off = ebase + ci * _CHUNK
        pltpu.sync_copy(rows.at[pl.ds(off, _CHUNK)], idxs[j])

    def _issue_ones(j):
        pltpu.async_copy(onesbuf, d_sp.at[idxs[j]], sems[j], add=True)

    def _wait_ones(j):
        pltpu.make_async_copy(onesbuf, d_sp.at[idxs[j]], sems[j]).wait()

    _load_ridx(0, 0)
    _load_ridx(1, 1)

    def _pair2(i, _):
        _issue_ones(0)
        _issue_ones(1)
        for j in (0, 1):
            p = jnp.minimum(2 * i + 2 + j, last)
            _wait_ones(j)
            _load_ridx(j, p)
        return 0
    lax.fori_loop(0, (n_chunks - 1) // 2, _pair2, 0)

    _issue_ones(0)
    _wait_ones(0)

    plsc.subcore_barrier()
    pltpu.sync_copy(d_sp.at[pl.ds(r0, _STRIPE)],
                    deg_hbm.at[c, pl.ds(r0, _STRIPE)])


def _sc_aggregate(edge_index, edge_attr, x, rel_embed):
    mesh = plsc.VectorSubcoreMesh(core_axis_name="c", subcore_axis_name="s")
    acc_fn = pl.kernel(
        _sc_acc_body,
        out_type=[
            jax.ShapeDtypeStruct((2, _NPAD, _D), jnp.float32),
        ],
        mesh=mesh,
        scratch_types=[
            pltpu.VMEM_SHARED((_NPAD, _D), jnp.float32),
            pltpu.VMEM_SHARED((_RPAD, _D), jnp.float32),
            pltpu.VMEM((_CHUNK,), jnp.int32),
            pltpu.VMEM((_CHUNK,), jnp.int32),
            pltpu.VMEM((_CHUNK,), jnp.int32),
            pltpu.VMEM((_CHUNK, _D), jnp.float32),
            pltpu.VMEM((_CHUNK, _D), jnp.float32),
            pltpu.VMEM((_CHUNK,), jnp.int32),
            pltpu.VMEM((_CHUNK,), jnp.int32),
            pltpu.VMEM((_CHUNK,), jnp.int32),
            pltpu.VMEM((_CHUNK, _D), jnp.float32),
            pltpu.VMEM((_CHUNK, _D), jnp.float32),
            pltpu.SemaphoreType.DMA,
            pltpu.SemaphoreType.DMA,
            pltpu.SemaphoreType.DMA,
            pltpu.SemaphoreType.DMA,
            pltpu.SemaphoreType.DMA,
            pltpu.SemaphoreType.DMA,
        ],
    )
    deg_fn = pl.kernel(
        _sc_deg_body,
        out_type=[
            jax.ShapeDtypeStruct((2, _NPAD, _D), jnp.float32),
        ],
        mesh=mesh,
        scratch_types=[
            pltpu.VMEM_SHARED((_NPAD, _D), jnp.float32),
            pltpu.VMEM((_CHUNK,), jnp.int32),
            pltpu.VMEM((_CHUNK,), jnp.int32),
            pltpu.VMEM((_CHUNK, _D), jnp.float32),
            pltpu.SemaphoreType.DMA,
            pltpu.SemaphoreType.DMA,
        ],
    )
    rel_padded = jnp.concatenate(
        [rel_embed, jnp.zeros((_RPAD - _R, _D), jnp.float32)], axis=0)
    acc, = acc_fn(edge_index.reshape(-1), edge_attr, x, rel_padded)
    deg, = deg_fn(edge_index[0])
    return acc, deg


_RB = 2000


def _tc_p1(acc_ref, x_ref, loop_rel_ref,
           w_in_ref, w_out_ref, w_loop_ref, deg_ref, bias_ref,
           h_ref, sums_ref):
    i = pl.program_id(0)
    hp = lax.Precision.HIGHEST
    m0 = jnp.dot(acc_ref[0], w_in_ref[...], precision=hp,
                 preferred_element_type=jnp.float32)
    m1 = jnp.dot(acc_ref[1], w_out_ref[...], precision=hp,
                 preferred_element_type=jnp.float32)
    xl = x_ref[...] * loop_rel_ref[0:1, :]
    m2 = jnp.dot(xl, w_loop_ref[...], precision=hp,
                 preferred_element_type=jnp.float32)
    inv_in = 1.0 / jnp.maximum(deg_ref[0, :, 0:1], 1.0)
    inv_out = 1.0 / jnp.maximum(deg_ref[1, :, 0:1], 1.0)
    h = m0 * inv_in + m1 * inv_out + m2
    h = h * (1.0 / 3.0) + bias_ref[...][None, :]
    h_ref[...] = h
    s1 = jnp.sum(h, axis=0, keepdims=True)
    s2 = jnp.sum(h * h, axis=0, keepdims=True)
    blk = jnp.concatenate([s1, s2, jnp.zeros((6, _D), jnp.float32)], axis=0)

    @pl.when(i == 0)
    def _():
        sums_ref[...] = blk

    @pl.when(i > 0)
    def _():
        sums_ref[...] = sums_ref[...] + blk


def _tc_bn(h_ref, sums_ref, gamma_ref, beta_ref, out_ref):
    inv_n = 1.0 / _N
    mean = sums_ref[0:1, :] * inv_n
    var = sums_ref[1:2, :] * inv_n - mean * mean
    scale = gamma_ref[...][None, :] * lax.rsqrt(var + 1e-5)
    out_ref[...] = (h_ref[...] - mean) * scale + beta_ref[...][None, :]


def _tc_rel(rel_ref, w_rel_ref, rel_out_ref):
    rel_out_ref[...] = jnp.dot(rel_ref[...], w_rel_ref[...],
                               precision=lax.Precision.HIGHEST,
                               preferred_element_type=jnp.float32)


def _tc_combine(acc, deg, x, rel_embed, loop_rel,
                weight_in, weight_out, weight_loop, weight_rel,
                bias, bn_gamma, bn_beta):
    nblk = _N // _RB
    full = lambda *shape: pl.BlockSpec(shape, lambda i: tuple(0 for _ in shape))
    h, sums = pl.pallas_call(
        _tc_p1,
        grid=(nblk,),
        in_specs=[
            pl.BlockSpec((2, _RB, _D), lambda i: (0, i, 0)),
            pl.BlockSpec((_RB, _D), lambda i: (i, 0)),
            full(1, _D),
            full(_D, _D),
            full(_D, _D),
            full(_D, _D),
            pl.BlockSpec((2, _RB, _D), lambda i: (0, i, 0)),
            full(_D),
        ],
        out_specs=[
            pl.BlockSpec((_RB, _D), lambda i: (i, 0)),
            pl.BlockSpec((8, _D), lambda i: (0, 0)),
        ],
        out_shape=[
            jax.ShapeDtypeStruct((_N, _D), jnp.float32),
            jax.ShapeDtypeStruct((8, _D), jnp.float32),
        ],
    )(acc, x, loop_rel, weight_in, weight_out, weight_loop, deg, bias)

    out = pl.pallas_call(
        _tc_bn,
        grid=(nblk,),
        in_specs=[
            pl.BlockSpec((_RB, _D), lambda i: (i, 0)),
            pl.BlockSpec((8, _D), lambda i: (0, 0)),
            full(_D),
            full(_D),
        ],
        out_specs=pl.BlockSpec((_RB, _D), lambda i: (i, 0)),
        out_shape=jax.ShapeDtypeStruct((_N, _D), jnp.float32),
    )(h, sums, bn_gamma, bn_beta)

    rel_out = pl.pallas_call(
        _tc_rel,
        out_shape=jax.ShapeDtypeStruct((_R, _D), jnp.float32),
    )(rel_embed, weight_rel)
    return out, rel_out


def kernel(x, rel_embed, edge_index, edge_attr, weight_in, weight_out,
           weight_rel, weight_loop, loop_rel, bias, bn_gamma, bn_beta):
    acc, deg = _sc_aggregate(edge_index, edge_attr, x, rel_embed)
    out, rel_out = _tc_combine(acc, deg, x, rel_embed, loop_rel,
                               weight_in, weight_out, weight_loop,
                               weight_rel, bias, bn_gamma, bn_beta)
    return out, rel_out


# speedup vs baseline: 1.1007x; 1.1007x over previous
import jax
import jax.numpy as jnp
from jax import lax
from jax.experimental import pallas as pl
from jax.experimental.pallas import tpu as pltpu
from jax.experimental.pallas import tpu_sc as plsc

_N = 10000
_NPAD = 10240
_D = 128
_R = 200
_RPAD = 256
_NSUB = 16
_CHUNK = 80
_STRIPE = _NPAD // _NSUB


_DW = 32


def _fill(buf, val, nsl=8):
    def body(i, _):
        r = i // nsl
        kk = (i % nsl) * 16
        buf[r, pl.ds(kk, 16)] = val
        return 0
    lax.fori_loop(0, _CHUNK * nsl, body, 0)


def _sc_acc_body(ei, ea, x_hbm, rel_hbm, acc_hbm,
                 a_sp, rel_sp,
                 ridx0, cidx0, tidx0, xbuf0, rbuf0,
                 ridx1, cidx1, tidx1, xbuf1, rbuf1,
                 gx0, gr0, sc0, gx1, gr1, sc1):
    c = lax.axis_index("c")
    s = lax.axis_index("s")
    num_e = ei.shape[0] // 2
    half = num_e // 2
    per_sub = half // _NSUB
    n_chunks = per_sub // _CHUNK
    last = n_chunks - 1
    zero16 = jnp.zeros((16,), jnp.float32)
    sets = ((ridx0, cidx0, tidx0, xbuf0, rbuf0, gx0, gr0, sc0),
            (ridx1, cidx1, tidx1, xbuf1, rbuf1, gx1, gr1, sc1))

    def _mul(xb, rb):
        def body(r4, _):
            r = r4 * 4
            for dr in range(4):
                for kk in range(8):
                    sl = pl.ds(kk * 16, 16)
                    xb[r + dr, sl] = xb[r + dr, sl] * rb[r + dr, sl]
            return 0
        lax.fori_loop(0, _CHUNK // 4, body, 0)

    def _load_idx(j, ci):
        ridx, cidx, tidx = sets[j][0], sets[j][1], sets[j][2]
        off = ebase + ci * _CHUNK
        pltpu.sync_copy(ei.at[pl.ds(off, _CHUNK)], ridx)
        pltpu.sync_copy(ei.at[pl.ds(num_e + off, _CHUNK)], cidx)
        pltpu.sync_copy(ea.at[pl.ds(off, _CHUNK)], tidx)

    def _issue_gathers(j):
        _, cidx, tidx, xb, rb, gx, gr, _sc = sets[j]
        pltpu.async_copy(x_hbm.at[cidx], xb, gx)
        pltpu.async_copy(rel_sp.at[tidx], rb, gr)

    def _wait_gathers(j):
        _, cidx, tidx, xb, rb, gx, gr, _sc = sets[j]
        pltpu.make_async_copy(x_hbm.at[cidx], xb, gx).wait()
        pltpu.make_async_copy(rel_sp.at[tidx], rb, gr).wait()

    def _issue_scatter(j):
        ridx, _, _, xb, _, _, _, sc = sets[j]
        pltpu.async_copy(xb, a_sp.at[ridx], sc, add=True)

    def _wait_scatter(j):
        ridx, _, _, xb, _, _, _, sc = sets[j]
        pltpu.make_async_copy(xb, a_sp.at[ridx], sc).wait()

    pltpu.sync_copy(rel_hbm.at[pl.ds(s * (_RPAD // _NSUB), _RPAD // _NSUB)],
                    rel_sp.at[pl.ds(s * (_RPAD // _NSUB), _RPAD // _NSUB)])
    _fill(xbuf0, zero16)
    r0 = s * _STRIPE
    for b in range(_STRIPE // _CHUNK):
        pltpu.sync_copy(xbuf0, a_sp.at[pl.ds(r0 + b * _CHUNK, _CHUNK)])
    plsc.subcore_barrier()

    ebase = c * half + s * per_sub

    _load_idx(0, 0)
    _issue_gathers(0)
    _load_idx(1, 1)
    _issue_gathers(1)

    def _pair(i, _):
        for j in (0, 1):
            _wait_gathers(j)
            _mul(sets[j][3], sets[j][4])
            _issue_scatter(j)
        for j in (0, 1):
            p = jnp.minimum(2 * i + 2 + j, last)
            _wait_scatter(j)
            _load_idx(j, p)
            _issue_gathers(j)
        return 0
    lax.fori_loop(0, (n_chunks - 1) // 2, _pair, 0)

    _wait_gathers(0)
    _mul(xbuf0, rbuf0)
    _issue_scatter(0)
    _wait_scatter(0)
    _wait_gathers(1)

    plsc.subcore_barrier()
    pltpu.sync_copy(a_sp.at[pl.ds(r0, _STRIPE)],
                    acc_hbm.at[c, pl.ds(r0, _STRIPE)])


def _sc_deg_body(rows, deg_hbm,
                 d_sp, ridx0, ridx1, onesbuf, s0, s1):
    c = lax.axis_index("c")
    s = lax.axis_index("s")
    half = rows.shape[0] // 2
    per_sub = half // _NSUB
    n_chunks = per_sub // _CHUNK
    last = n_chunks - 1
    zero16 = jnp.zeros((16,), jnp.float32)
    one16 = jnp.ones((16,), jnp.float32)
    sems = (s0, s1)
    idxs = (ridx0, ridx1)

    _fill(onesbuf, zero16, _DW // 16)
    r0 = s * _STRIPE
    for b in range(_STRIPE // _CHUNK):
        pltpu.sync_copy(onesbuf, d_sp.at[pl.ds(r0 + b * _CHUNK, _CHUNK)])
    _fill(onesbuf, one16, _DW // 16)
    plsc.subcore_barrier()

    ebase = c * half + s * per_sub

    def _load_ridx(j, ci):
        off = ebase + ci * _CHUNK
        pltpu.sync_copy(rows.at[pl.ds(off, _CHUNK)], idxs[j])

    def _issue_ones(j):
        pltpu.async_copy(onesbuf, d_sp.at[idxs[j]], sems[j], add=True)

    def _wait_ones(j):
        pltpu.make_async_copy(onesbuf, d_sp.at[idxs[j]], sems[j]).wait()

    _load_ridx(0, 0)
    _load_ridx(1, 1)

    def _pair2(i, _):
        _issue_ones(0)
        _issue_ones(1)
        for j in (0, 1):
            p = jnp.minimum(2 * i + 2 + j, last)
            _wait_ones(j)
            _load_ridx(j, p)
        return 0
    lax.fori_loop(0, (n_chunks - 1) // 2, _pair2, 0)

    _issue_ones(0)
    _wait_ones(0)

    plsc.subcore_barrier()
    pltpu.sync_copy(d_sp.at[pl.ds(r0, _STRIPE)],
                    deg_hbm.at[c, pl.ds(r0, _STRIPE)])


def _sc_aggregate(edge_index, edge_attr, x, rel_embed):
    mesh = plsc.VectorSubcoreMesh(core_axis_name="c", subcore_axis_name="s")
    acc_fn = pl.kernel(
        _sc_acc_body,
        out_type=[
            jax.ShapeDtypeStruct((2, _NPAD, _D), jnp.float32),
        ],
        mesh=mesh,
        scratch_types=[
            pltpu.VMEM_SHARED((_NPAD, _D), jnp.float32),
            pltpu.VMEM_SHARED((_RPAD, _D), jnp.float32),
            pltpu.VMEM((_CHUNK,), jnp.int32),
            pltpu.VMEM((_CHUNK,), jnp.int32),
            pltpu.VMEM((_CHUNK,), jnp.int32),
            pltpu.VMEM((_CHUNK, _D), jnp.float32),
            pltpu.VMEM((_CHUNK, _D), jnp.float32),
            pltpu.VMEM((_CHUNK,), jnp.int32),
            pltpu.VMEM((_CHUNK,), jnp.int32),
            pltpu.VMEM((_CHUNK,), jnp.int32),
            pltpu.VMEM((_CHUNK, _D), jnp.float32),
            pltpu.VMEM((_CHUNK, _D), jnp.float32),
            pltpu.SemaphoreType.DMA,
            pltpu.SemaphoreType.DMA,
            pltpu.SemaphoreType.DMA,
            pltpu.SemaphoreType.DMA,
            pltpu.SemaphoreType.DMA,
            pltpu.SemaphoreType.DMA,
        ],
    )
    deg_fn = pl.kernel(
        _sc_deg_body,
        out_type=[
            jax.ShapeDtypeStruct((2, _NPAD, _DW), jnp.float32),
        ],
        mesh=mesh,
        scratch_types=[
            pltpu.VMEM_SHARED((_NPAD, _DW), jnp.float32),
            pltpu.VMEM((_CHUNK,), jnp.int32),
            pltpu.VMEM((_CHUNK,), jnp.int32),
            pltpu.VMEM((_CHUNK, _DW), jnp.float32),
            pltpu.SemaphoreType.DMA,
            pltpu.SemaphoreType.DMA,
        ],
    )
    rel_padded = jnp.concatenate(
        [rel_embed, jnp.zeros((_RPAD - _R, _D), jnp.float32)], axis=0)
    acc, = acc_fn(edge_index.reshape(-1), edge_attr, x, rel_padded)
    deg, = deg_fn(edge_index[0])
    return acc, deg


_RB = 2000


def _tc_mm(acc_ref, x_ref, loop_rel_ref,
           w_in_ref, w_out_ref, w_loop_ref, m_ref):
    hp = lax.Precision.HIGHEST
    m_ref[0] = jnp.dot(acc_ref[0], w_in_ref[...], precision=hp,
                       preferred_element_type=jnp.float32)
    m_ref[1] = jnp.dot(acc_ref[1], w_out_ref[...], precision=hp,
                       preferred_element_type=jnp.float32)
    xl = x_ref[...] * loop_rel_ref[0:1, :]
    m_ref[2] = jnp.dot(xl, w_loop_ref[...], precision=hp,
                       preferred_element_type=jnp.float32)


def _tc_norm(m_ref, deg_ref, bias_ref, h_ref, sums_ref):
    i = pl.program_id(0)
    inv_in = 1.0 / jnp.maximum(deg_ref[0, :, 0:1], 1.0)
    inv_out = 1.0 / jnp.maximum(deg_ref[1, :, 0:1], 1.0)
    h = m_ref[0] * inv_in + m_ref[1] * inv_out + m_ref[2]
    h = h * (1.0 / 3.0) + bias_ref[...][None, :]
    h_ref[...] = h
    s1 = jnp.sum(h, axis=0, keepdims=True)
    s2 = jnp.sum(h * h, axis=0, keepdims=True)
    blk = jnp.concatenate([s1, s2, jnp.zeros((6, _D), jnp.float32)], axis=0)

    @pl.when(i == 0)
    def _():
        sums_ref[...] = blk

    @pl.when(i > 0)
    def _():
        sums_ref[...] = sums_ref[...] + blk


def _tc_bn(h_ref, sums_ref, gamma_ref, beta_ref, out_ref):
    inv_n = 1.0 / _N
    mean = sums_ref[0:1, :] * inv_n
    var = sums_ref[1:2, :] * inv_n - mean * mean
    scale = gamma_ref[...][None, :] * lax.rsqrt(var + 1e-5)
    out_ref[...] = (h_ref[...] - mean) * scale + beta_ref[...][None, :]


def _tc_rel(rel_ref, w_rel_ref, rel_out_ref):
    rel_out_ref[...] = jnp.dot(rel_ref[...], w_rel_ref[...],
                               precision=lax.Precision.HIGHEST,
                               preferred_element_type=jnp.float32)


def _tc_combine(acc, deg, x, rel_embed, loop_rel,
                weight_in, weight_out, weight_loop, weight_rel,
                bias, bn_gamma, bn_beta):
    nblk = _N // _RB
    full = lambda *shape: pl.BlockSpec(shape, lambda i: tuple(0 for _ in shape))
    m = pl.pallas_call(
        _tc_mm,
        grid=(nblk,),
        in_specs=[
            pl.BlockSpec((2, _RB, _D), lambda i: (0, i, 0)),
            pl.BlockSpec((_RB, _D), lambda i: (i, 0)),
            full(1, _D),
            full(_D, _D),
            full(_D, _D),
            full(_D, _D),
        ],
        out_specs=pl.BlockSpec((3, _RB, _D), lambda i: (0, i, 0)),
        out_shape=jax.ShapeDtypeStruct((3, _N, _D), jnp.float32),
    )(acc, x, loop_rel, weight_in, weight_out, weight_loop)

    h, sums = pl.pallas_call(
        _tc_norm,
        grid=(nblk,),
        in_specs=[
            pl.BlockSpec((3, _RB, _D), lambda i: (0, i, 0)),
            pl.BlockSpec((2, _RB, _DW), lambda i: (0, i, 0)),
            full(_D),
        ],
        out_specs=[
            pl.BlockSpec((_RB, _D), lambda i: (i, 0)),
            pl.BlockSpec((8, _D), lambda i: (0, 0)),
        ],
        out_shape=[
            jax.ShapeDtypeStruct((_N, _D), jnp.float32),
            jax.ShapeDtypeStruct((8, _D), jnp.float32),
        ],
    )(m, deg, bias)

    out = pl.pallas_call(
        _tc_bn,
        grid=(nblk,),
        in_specs=[
            pl.BlockSpec((_RB, _D), lambda i: (i, 0)),
            pl.BlockSpec((8, _D), lambda i: (0, 0)),
            full(_D),
            full(_D),
        ],
        out_specs=pl.BlockSpec((_RB, _D), lambda i: (i, 0)),
        out_shape=jax.ShapeDtypeStruct((_N, _D), jnp.float32),
    )(h, sums, bn_gamma, bn_beta)

    rel_out = pl.pallas_call(
        _tc_rel,
        out_shape=jax.ShapeDtypeStruct((_R, _D), jnp.float32),
    )(rel_embed, weight_rel)
    return out, rel_out


def kernel(x, rel_embed, edge_index, edge_attr, weight_in, weight_out,
           weight_rel, weight_loop, loop_rel, bias, bn_gamma, bn_beta):
    acc, deg = _sc_aggregate(edge_index, edge_attr, x, rel_embed)
    out, rel_out = _tc_combine(acc, deg, x, rel_embed, loop_rel,
                               weight_in, weight_out, weight_loop,
                               weight_rel, bias, bn_gamma, bn_beta)
    return out, rel_out
